# baseline (speedup 1.0000x reference)
import functools

import jax
import jax.numpy as jnp
from jax import lax
from jax.experimental import pallas as pl
from jax.experimental.pallas import tpu as pltpu
from jax.experimental.pallas import tpu_sc as plsc

VOCAB = 100000
D = 128
C = 128
B = 4096
L = 200

NC = 2
NS = 16
NW = NC * NS
RPW = B // NW
CHUNK = 50
CPR = L // CHUNK
NCHUNK = RPW * CPR
NVR = D // 16
NBUF = 8
RPI = NBUF // CPR
NSEG = 4
SEG_ROWS = RPW // NSEG
SEG_ITERS = NCHUNK // NBUF // NSEG


def _accum_chunk(buf, acc):
  def body(j, a):
    return tuple(a[k] + buf[j, pl.ds(16 * k, 16)] for k in range(NVR))
  return lax.fori_loop(0, CHUNK, body, acc, unroll=10)


def _sc_body(table, idxh, outh, idx_v, bufs, outs, sems, osems):
  cid = lax.axis_index("c")
  sid = lax.axis_index("s")
  wid = sid * NC + cid
  pltpu.sync_copy(idxh.at[pl.ds(wid * NCHUNK, NCHUNK)], idx_v)

  def idx_ref(chunk_row, bb):
    return idx_v.at[chunk_row * CPR + (bb % CPR)]

  for bb in range(NBUF):
    pltpu.async_copy(table.at[idx_ref(bb // CPR, bb)], bufs[bb], sems[bb])

  zeros = tuple(jnp.zeros((16,), jnp.float32) for _ in range(NVR))
  inv_l = jnp.float32(1.0 / L)

  for seg in range(NSEG):
    par = seg % 2
    if seg >= 2:
      pltpu.make_async_copy(
          outs[par], outh.at[pl.ds((wid * NSEG + seg - 2) * SEG_ROWS,
                                   SEG_ROWS)], osems[par]).wait()

    def iter_body(i, carry, seg=seg, par=par):
      row0 = RPI * i + seg * SEG_ITERS * RPI
      rloc = RPI * i
      for bb in range(NBUF):
        pltpu.make_async_copy(table.at[idx_ref(0, bb)], bufs[bb],
                              sems[bb]).wait()
        acc = _accum_chunk(bufs[bb], zeros if bb % CPR == 0 else acc)
        nrow = jnp.minimum(row0 + bb // CPR + RPI, RPW - RPI + bb // CPR)
        pltpu.async_copy(table.at[idx_ref(nrow, bb)], bufs[bb], sems[bb])
        if bb % CPR == CPR - 1:
          row = rloc + bb // CPR
          for k in range(NVR):
            outs[par][row, pl.ds(16 * k, 16)] = acc[k] * inv_l
      return carry

    lax.fori_loop(0, SEG_ITERS, iter_body, 0)
    pltpu.async_copy(
        outs[par], outh.at[pl.ds((wid * NSEG + seg) * SEG_ROWS, SEG_ROWS)],
        osems[par])

  for bb in range(NBUF):
    pltpu.make_async_copy(table.at[idx_ref(0, bb)], bufs[bb], sems[bb]).wait()
  for seg in (NSEG - 2, NSEG - 1):
    par = seg % 2
    pltpu.make_async_copy(
        outs[par], outh.at[pl.ds((wid * NSEG + seg) * SEG_ROWS, SEG_ROWS)],
        osems[par]).wait()


def _sc_pool(table, ids):
  mesh = plsc.VectorSubcoreMesh(core_axis_name="c", subcore_axis_name="s")
  return pl.kernel(
      _sc_body,
      out_type=jax.ShapeDtypeStruct((B, D), jnp.float32),
      mesh=mesh,
      scratch_types=[
          pltpu.VMEM((NCHUNK, CHUNK), jnp.int32),
          [pltpu.VMEM((CHUNK, D), jnp.float32) for _ in range(NBUF)],
          [pltpu.VMEM((SEG_ROWS, D), jnp.float32) for _ in range(2)],
          [pltpu.SemaphoreType.DMA for _ in range(NBUF)],
          [pltpu.SemaphoreType.DMA for _ in range(2)],
      ],
  )(table, ids)


def _head_body(x_ref, w_ref, b_ref, g_ref, be_ref, lab_ref,
               loss_ref, preds_ref):
  x = x_ref[...]
  w = w_ref[...]
  y = lax.dot_general(x, w, (((1,), (1,)), ((), ())),
                      preferred_element_type=jnp.float32) + b_ref[...]
  mu = jnp.mean(y, axis=-1, keepdims=True)
  d = y - mu
  var = jnp.mean(d * d, axis=-1, keepdims=True)
  xn = d * lax.rsqrt(var + 1e-5) * g_ref[...] + be_ref[...]
  m = jnp.max(xn, axis=-1, keepdims=True)
  e = jnp.exp(xn - m)
  lse = jnp.log(jnp.sum(e, axis=-1, keepdims=True)) + m
  col = lax.broadcasted_iota(jnp.int32, (B, C), 1)
  picked = jnp.sum(jnp.where(col == lab_ref[...], xn, 0.0),
                   axis=-1, keepdims=True)
  loss_ref[...] = jnp.sum(lse - picked, axis=(0, 1), keepdims=True) * (1.0 / B)
  preds_ref[...] = jnp.argmax(xn, axis=-1).astype(jnp.int32)


def _head(pooled, label_ids, W, b, gamma, beta):
  return pl.pallas_call(
      _head_body,
      out_shape=(
          jax.ShapeDtypeStruct((1, 1), jnp.float32),
          jax.ShapeDtypeStruct((B,), jnp.int32),
      ),
  )(pooled, W, b.reshape(1, C), gamma.reshape(1, C), beta.reshape(1, C),
    label_ids.reshape(B, 1).astype(jnp.int32))


@jax.jit
def kernel(word_ids, label_ids, embed_table, W, b, gamma, beta):
  idx2d = word_ids.astype(jnp.int32).reshape(B * CPR, CHUNK)
  pooled = _sc_pool(embed_table, idx2d)
  loss, preds = _head(pooled, label_ids, W, b, gamma, beta)
  return loss[0, 0], preds


# speedup vs baseline: 1.0025x; 1.0025x over previous
import functools

import jax
import jax.numpy as jnp
from jax import lax
from jax.experimental import pallas as pl
from jax.experimental.pallas import tpu as pltpu
from jax.experimental.pallas import tpu_sc as plsc

VOCAB = 100000
D = 128
C = 128
B = 4096
L = 200

NC = 2
NS = 16
NW = NC * NS
RPW = B // NW
CHUNK = 100
CPR = L // CHUNK
NCHUNK = RPW * CPR
NVR = D // 16
NBUF = 4
RPI = NBUF // CPR
NSEG = 4
SEG_ROWS = RPW // NSEG
SEG_ITERS = NCHUNK // NBUF // NSEG


def _accum_chunk(buf, acc):
  def body(j, a):
    return tuple(a[k] + buf[j, pl.ds(16 * k, 16)] for k in range(NVR))
  return lax.fori_loop(0, CHUNK, body, acc, unroll=10)


def _sc_body(table, idxh, outh, idx_v, bufs, outs, sems, osems):
  cid = lax.axis_index("c")
  sid = lax.axis_index("s")
  wid = sid * NC + cid
  pltpu.sync_copy(idxh.at[pl.ds(wid * NCHUNK, NCHUNK)], idx_v)

  def idx_ref(chunk_row, bb):
    return idx_v.at[chunk_row * CPR + (bb % CPR)]

  for bb in range(NBUF):
    pltpu.async_copy(table.at[idx_ref(bb // CPR, bb)], bufs[bb], sems[bb])

  zeros = tuple(jnp.zeros((16,), jnp.float32) for _ in range(NVR))
  inv_l = jnp.float32(1.0 / L)

  for seg in range(NSEG):
    par = seg % 2
    if seg >= 2:
      pltpu.make_async_copy(
          outs[par], outh.at[pl.ds((wid * NSEG + seg - 2) * SEG_ROWS,
                                   SEG_ROWS)], osems[par]).wait()

    def iter_body(i, carry, seg=seg, par=par):
      row0 = RPI * i + seg * SEG_ITERS * RPI
      rloc = RPI * i
      for bb in range(NBUF):
        pltpu.make_async_copy(table.at[idx_ref(0, bb)], bufs[bb],
                              sems[bb]).wait()
        acc = _accum_chunk(bufs[bb], zeros if bb % CPR == 0 else acc)
        nrow = jnp.minimum(row0 + bb // CPR + RPI, RPW - RPI + bb // CPR)
        pltpu.async_copy(table.at[idx_ref(nrow, bb)], bufs[bb], sems[bb])
        if bb % CPR == CPR - 1:
          row = rloc + bb // CPR
          for k in range(NVR):
            outs[par][row, pl.ds(16 * k, 16)] = acc[k] * inv_l
      return carry

    lax.fori_loop(0, SEG_ITERS, iter_body, 0)
    pltpu.async_copy(
        outs[par], outh.at[pl.ds((wid * NSEG + seg) * SEG_ROWS, SEG_ROWS)],
        osems[par])

  for bb in range(NBUF):
    pltpu.make_async_copy(table.at[idx_ref(0, bb)], bufs[bb], sems[bb]).wait()
  for seg in (NSEG - 2, NSEG - 1):
    par = seg % 2
    pltpu.make_async_copy(
        outs[par], outh.at[pl.ds((wid * NSEG + seg) * SEG_ROWS, SEG_ROWS)],
        osems[par]).wait()


def _sc_pool(table, ids):
  mesh = plsc.VectorSubcoreMesh(core_axis_name="c", subcore_axis_name="s")
  return pl.kernel(
      _sc_body,
      out_type=jax.ShapeDtypeStruct((B, D), jnp.float32),
      mesh=mesh,
      scratch_types=[
          pltpu.VMEM((NCHUNK, CHUNK), jnp.int32),
          [pltpu.VMEM((CHUNK, D), jnp.float32) for _ in range(NBUF)],
          [pltpu.VMEM((SEG_ROWS, D), jnp.float32) for _ in range(2)],
          [pltpu.SemaphoreType.DMA for _ in range(NBUF)],
          [pltpu.SemaphoreType.DMA for _ in range(2)],
      ],
  )(table, ids)


def _head_body(x_ref, w_ref, b_ref, g_ref, be_ref, lab_ref,
               loss_ref, preds_ref):
  x = x_ref[...]
  w = w_ref[...]
  y = lax.dot_general(x, w, (((1,), (1,)), ((), ())),
                      preferred_element_type=jnp.float32) + b_ref[...]
  mu = jnp.mean(y, axis=-1, keepdims=True)
  d = y - mu
  var = jnp.mean(d * d, axis=-1, keepdims=True)
  xn = d * lax.rsqrt(var + 1e-5) * g_ref[...] + be_ref[...]
  m = jnp.max(xn, axis=-1, keepdims=True)
  e = jnp.exp(xn - m)
  lse = jnp.log(jnp.sum(e, axis=-1, keepdims=True)) + m
  col = lax.broadcasted_iota(jnp.int32, (B, C), 1)
  picked = jnp.sum(jnp.where(col == lab_ref[...], xn, 0.0),
                   axis=-1, keepdims=True)
  loss_ref[...] = jnp.sum(lse - picked, axis=(0, 1), keepdims=True) * (1.0 / B)
  preds_ref[...] = jnp.argmax(xn, axis=-1).astype(jnp.int32)


def _head(pooled, label_ids, W, b, gamma, beta):
  return pl.pallas_call(
      _head_body,
      out_shape=(
          jax.ShapeDtypeStruct((1, 1), jnp.float32),
          jax.ShapeDtypeStruct((B,), jnp.int32),
      ),
  )(pooled, W, b.reshape(1, C), gamma.reshape(1, C), beta.reshape(1, C),
    label_ids.reshape(B, 1).astype(jnp.int32))


@jax.jit
def kernel(word_ids, label_ids, embed_table, W, b, gamma, beta):
  idx2d = word_ids.astype(jnp.int32).reshape(B * CPR, CHUNK)
  pooled = _sc_pool(embed_table, idx2d)
  loss, preds = _head(pooled, label_ids, W, b, gamma, beta)
  return loss[0, 0], preds


# speedup vs baseline: 1.0052x; 1.0027x over previous
import functools

import jax
import jax.numpy as jnp
from jax import lax
from jax.experimental import pallas as pl
from jax.experimental.pallas import tpu as pltpu
from jax.experimental.pallas import tpu_sc as plsc

VOCAB = 100000
D = 128
C = 128
B = 4096
L = 200

NC = 2
NS = 16
NW = NC * NS
RPW = B // NW
CHUNK = 100
CPR = L // CHUNK
NCHUNK = RPW * CPR
NVR = D // 16
NBUF = 4
RPI = NBUF // CPR
NSEG = 4
SEG_ROWS = RPW // NSEG
SEG_ITERS = NCHUNK // NBUF // NSEG


def _accum_chunk(buf, acc):
  def body(j, a):
    return tuple(a[k] + buf[j, pl.ds(16 * k, 16)] for k in range(NVR))
  return lax.fori_loop(0, CHUNK, body, acc, unroll=10)


def _sc_body(table, idxh, outh, idx_v, bufs, outs, sems, osems):
  cid = lax.axis_index("c")
  sid = lax.axis_index("s")
  wid = sid * NC + cid
  pltpu.sync_copy(idxh.at[pl.ds(wid * NCHUNK, NCHUNK)], idx_v)

  def idx_ref(chunk_row, bb):
    return idx_v.at[chunk_row * CPR + (bb % CPR)]

  for bb in range(NBUF):
    pltpu.async_copy(table.at[idx_ref(bb // CPR, bb)], bufs[bb], sems[bb])

  zeros = tuple(jnp.zeros((16,), jnp.float32) for _ in range(NVR))
  inv_l = jnp.float32(1.0 / L)

  for seg in range(NSEG):
    par = seg % 2
    if seg >= 2:
      pltpu.make_async_copy(
          outs[par], outh.at[pl.ds((wid * NSEG + seg - 2) * SEG_ROWS,
                                   SEG_ROWS)], osems[par]).wait()

    def iter_body(i, carry, seg=seg, par=par):
      row0 = RPI * i + seg * SEG_ITERS * RPI
      rloc = RPI * i
      for bb in range(NBUF):
        pltpu.make_async_copy(table.at[idx_ref(0, bb)], bufs[bb],
                              sems[bb]).wait()
        acc = _accum_chunk(bufs[bb], zeros if bb % CPR == 0 else acc)
        pltpu.async_copy(table.at[idx_ref(row0 + bb // CPR + RPI, bb)],
                         bufs[bb], sems[bb])
        if bb % CPR == CPR - 1:
          row = rloc + bb // CPR
          for k in range(NVR):
            outs[par][row, pl.ds(16 * k, 16)] = acc[k] * inv_l
      return carry

    niter = SEG_ITERS if seg < NSEG - 1 else SEG_ITERS - 1
    lax.fori_loop(0, niter, iter_body, 0)
    if seg == NSEG - 1:
      rloc = RPI * (SEG_ITERS - 1)
      for bb in range(NBUF):
        pltpu.make_async_copy(table.at[idx_ref(0, bb)], bufs[bb],
                              sems[bb]).wait()
        acc = _accum_chunk(bufs[bb], zeros if bb % CPR == 0 else acc)
        if bb % CPR == CPR - 1:
          row = rloc + bb // CPR
          for k in range(NVR):
            outs[par][row, pl.ds(16 * k, 16)] = acc[k] * inv_l
    pltpu.async_copy(
        outs[par], outh.at[pl.ds((wid * NSEG + seg) * SEG_ROWS, SEG_ROWS)],
        osems[par])

  for seg in (NSEG - 2, NSEG - 1):
    par = seg % 2
    pltpu.make_async_copy(
        outs[par], outh.at[pl.ds((wid * NSEG + seg) * SEG_ROWS, SEG_ROWS)],
        osems[par]).wait()


def _sc_pool(table, ids):
  mesh = plsc.VectorSubcoreMesh(core_axis_name="c", subcore_axis_name="s")
  return pl.kernel(
      _sc_body,
      out_type=jax.ShapeDtypeStruct((B, D), jnp.float32),
      mesh=mesh,
      scratch_types=[
          pltpu.VMEM((NCHUNK, CHUNK), jnp.int32),
          [pltpu.VMEM((CHUNK, D), jnp.float32) for _ in range(NBUF)],
          [pltpu.VMEM((SEG_ROWS, D), jnp.float32) for _ in range(2)],
          [pltpu.SemaphoreType.DMA for _ in range(NBUF)],
          [pltpu.SemaphoreType.DMA for _ in range(2)],
      ],
  )(table, ids)


def _head_body(x_ref, w_ref, b_ref, g_ref, be_ref, lab_ref,
               loss_ref, preds_ref):
  x = x_ref[...]
  w = w_ref[...]
  y = lax.dot_general(x, w, (((1,), (1,)), ((), ())),
                      preferred_element_type=jnp.float32) + b_ref[...]
  mu = jnp.mean(y, axis=-1, keepdims=True)
  d = y - mu
  var = jnp.mean(d * d, axis=-1, keepdims=True)
  xn = d * lax.rsqrt(var + 1e-5) * g_ref[...] + be_ref[...]
  m = jnp.max(xn, axis=-1, keepdims=True)
  e = jnp.exp(xn - m)
  lse = jnp.log(jnp.sum(e, axis=-1, keepdims=True)) + m
  col = lax.broadcasted_iota(jnp.int32, (B, C), 1)
  picked = jnp.sum(jnp.where(col == lab_ref[...], xn, 0.0),
                   axis=-1, keepdims=True)
  loss_ref[...] = jnp.sum(lse - picked, axis=(0, 1), keepdims=True) * (1.0 / B)
  preds_ref[...] = jnp.argmax(xn, axis=-1).astype(jnp.int32)


def _head(pooled, label_ids, W, b, gamma, beta):
  return pl.pallas_call(
      _head_body,
      out_shape=(
          jax.ShapeDtypeStruct((1, 1), jnp.float32),
          jax.ShapeDtypeStruct((B,), jnp.int32),
      ),
  )(pooled, W, b.reshape(1, C), gamma.reshape(1, C), beta.reshape(1, C),
    label_ids.reshape(B, 1).astype(jnp.int32))


@jax.jit
def kernel(word_ids, label_ids, embed_table, W, b, gamma, beta):
  idx2d = word_ids.astype(jnp.int32).reshape(B * CPR, CHUNK)
  pooled = _sc_pool(embed_table, idx2d)
  loss, preds = _head(pooled, label_ids, W, b, gamma, beta)
  return loss[0, 0], preds
